# baseline (speedup 1.0000x reference)
import functools

import jax
import jax.numpy as jnp
from jax import lax
from jax.experimental import pallas as pl
from jax.experimental.pallas import tpu as pltpu
from jax.experimental.pallas import tpu_sc as plsc


def _routing_body(tb_ref, h_ref, wq_ref, wk_ref,
                  omp_ref, p_ref, mask_ref, sel_ref, carry_ref, *, block_rows):
    i = pl.program_id(0)
    bs = block_rows
    h = h_ref[...].astype(jnp.bfloat16)

    prev = carry_ref[...]
    hs = jnp.concatenate([prev, h[:-1, :]], axis=0)
    carry_ref[...] = h[bs - 1:bs, :]

    qst = jax.lax.dot_general(wq_ref[...], hs, (((1,), (1,)), ((), ())),
                              preferred_element_type=jnp.float32)
    kt = jax.lax.dot_general(wk_ref[...], h, (((1,), (1,)), ((), ())),
                             preferred_element_type=jnp.float32)

    qq = jnp.sum(qst * qst, axis=0, keepdims=True)
    kk = jnp.sum(kt * kt, axis=0, keepdims=True)
    qk = jnp.sum(qst * kt, axis=0, keepdims=True)

    denom = (jnp.maximum(jnp.sqrt(qq), 1e-12) *
             jnp.maximum(jnp.sqrt(kk), 1e-12))
    cs = qk / denom
    temp = jnp.clip(jnp.abs(tb_ref[0]), 0.1, 2.0)
    bias = tb_ref[1]
    p = jax.nn.sigmoid((1.0 - cs + bias) / temp)

    gidx = jax.lax.broadcasted_iota(jnp.int32, (1, bs), 1) + i * bs
    p = jnp.where(gidx == 0, 1.0, p)

    omp = 1.0 - p
    omp_ref[...] = omp
    p_ref[...] = p
    m = p > omp
    mask_ref[...] = m.astype(jnp.float32)
    sel_ref[...] = jnp.where(m, p, omp)


def _tc_stage(hidden_states, tb, Wq, Wk):
    T, D = hidden_states.shape
    BS = 4096
    grid_spec = pltpu.PrefetchScalarGridSpec(
        num_scalar_prefetch=1,
        grid=(T // BS,),
        in_specs=[
            pl.BlockSpec((BS, D), lambda i, *_: (i, 0)),
            pl.BlockSpec((D, D), lambda i, *_: (0, 0)),
            pl.BlockSpec((D, D), lambda i, *_: (0, 0)),
        ],
        out_specs=[
            pl.BlockSpec((1, BS), lambda i, *_: (0, i)),
            pl.BlockSpec((1, BS), lambda i, *_: (0, i)),
            pl.BlockSpec((1, BS), lambda i, *_: (0, i)),
            pl.BlockSpec((1, BS), lambda i, *_: (0, i)),
        ],
        scratch_shapes=[pltpu.VMEM((1, D), jnp.bfloat16)],
    )
    return pl.pallas_call(
        functools.partial(_routing_body, block_rows=BS),
        grid_spec=grid_spec,
        out_shape=[
            jax.ShapeDtypeStruct((1, T), jnp.float32),
            jax.ShapeDtypeStruct((1, T), jnp.float32),
            jax.ShapeDtypeStruct((1, T), jnp.float32),
            jax.ShapeDtypeStruct((1, T), jnp.float32),
        ],
        compiler_params=pltpu.CompilerParams(
            dimension_semantics=("arbitrary",)),
    )(tb, hidden_states, Wq, Wk)


def _sc_stage(omp, p, m, s, cu):
    T = omp.shape[0]
    n_workers = 32
    chunk = T // n_workers
    mesh = plsc.VectorSubcoreMesh(core_axis_name="c", subcore_axis_name="s")

    @functools.partial(
        pl.kernel, mesh=mesh,
        out_type=[jax.ShapeDtypeStruct((T + 16,), jnp.float32)] * 4,
        scratch_types=[pltpu.VMEM((16,), jnp.int32),
                       pltpu.VMEM((16,), jnp.float32),
                       pltpu.VMEM((16,), jnp.float32),
                       pltpu.SemaphoreType.DMA],
    )
    def sc_scatter(omp_in, p_in, m_in, s_in, cu_hbm,
                   omp_out, p_out, m_out, s_out,
                   cu_v, ones_v, zeros_v, sem):
        wid = lax.axis_index("c") * 16 + lax.axis_index("s")
        base = wid * chunk
        pltpu.sync_copy(cu_hbm.at[pl.ds(0, 16)], cu_v)
        gidx = cu_v[...]
        msk = jnp.logical_and(gidx >= base, gidx < base + chunk)
        idx_safe = jnp.where(msk, gidx, T + lax.iota(jnp.int32, 16))
        ones_v[...] = jnp.ones((16,), jnp.float32)
        zeros_v[...] = jnp.zeros((16,), jnp.float32)
        for src, dst, val in ((omp_in, omp_out, zeros_v), (p_in, p_out, ones_v),
                              (m_in, m_out, ones_v), (s_in, s_out, ones_v)):
            pltpu.sync_copy(src.at[pl.ds(base, chunk)], dst.at[pl.ds(base, chunk)])
            pltpu.async_copy(val, dst.at[idx_safe], sem).wait()

    out = sc_scatter(omp, p, m, s, cu)
    return tuple(o[:T] for o in out)


def kernel(hidden_states, cu_seqlens, Wq, Wk, temperature, boundary_bias):
    T, D = hidden_states.shape
    tb = jnp.stack([temperature.astype(jnp.float32),
                    boundary_bias.astype(jnp.float32)])
    omp_t, p_t, mask_t, sel_t = _tc_stage(hidden_states, tb,
                                          Wq.astype(jnp.bfloat16),
                                          Wk.astype(jnp.bfloat16))
    omp, p, m, s = _sc_stage(omp_t.reshape(T), p_t.reshape(T),
                             mask_t.reshape(T), sel_t.reshape(T),
                             cu_seqlens)
    prob = jnp.concatenate([omp[:, None], p[:, None]], axis=1)
    return (prob, m.astype(bool), s.reshape(T, 1))


# speedup vs baseline: 2.1234x; 2.1234x over previous
import functools

import jax
import jax.numpy as jnp
from jax.experimental import pallas as pl
from jax.experimental.pallas import tpu as pltpu


def _routing_body(cu_ref, tb_ref, h_ref, wq_ref, wk_ref,
                  prob_ref, mask_ref, sel_ref, carry_ref, *, block_rows):
    i = pl.program_id(0)
    bs = block_rows
    h = h_ref[...].astype(jnp.bfloat16)

    prev = carry_ref[...]
    hs = jnp.concatenate([prev, h[:-1, :]], axis=0)
    carry_ref[...] = h[bs - 1:bs, :]

    qst = jax.lax.dot_general(wq_ref[...], hs, (((1,), (1,)), ((), ())),
                              preferred_element_type=jnp.float32)
    kt = jax.lax.dot_general(wk_ref[...], h, (((1,), (1,)), ((), ())),
                             preferred_element_type=jnp.float32)

    qq = jnp.sum(qst * qst, axis=0, keepdims=True)
    kk = jnp.sum(kt * kt, axis=0, keepdims=True)
    qk = jnp.sum(qst * kt, axis=0, keepdims=True)

    denom = (jnp.maximum(jnp.sqrt(qq), 1e-12) *
             jnp.maximum(jnp.sqrt(kk), 1e-12))
    cs = qk / denom
    temp = jnp.clip(jnp.abs(tb_ref[0]), 0.1, 2.0)
    bias = tb_ref[1]
    p = jax.nn.sigmoid((1.0 - cs + bias) / temp)

    gidx = jax.lax.broadcasted_iota(jnp.int32, (1, bs), 1) + i * bs
    force = gidx == 0
    for j in range(16):
        force = jnp.logical_or(force, gidx == cu_ref[j])
    p = jnp.where(force, 1.0, p)

    omp = 1.0 - p
    prob_ref[...] = jnp.concatenate([omp, p], axis=0)
    m = p > omp
    mask_ref[...] = m.astype(jnp.float32)
    sel_ref[...] = jnp.where(m, p, omp)


def kernel(hidden_states, cu_seqlens, Wq, Wk, temperature, boundary_bias):
    T, D = hidden_states.shape
    BS = 4096
    tb = jnp.stack([temperature.astype(jnp.float32),
                    boundary_bias.astype(jnp.float32)])
    Wq = Wq.astype(jnp.bfloat16)
    Wk = Wk.astype(jnp.bfloat16)
    grid_spec = pltpu.PrefetchScalarGridSpec(
        num_scalar_prefetch=2,
        grid=(T // BS,),
        in_specs=[
            pl.BlockSpec((BS, D), lambda i, *_: (i, 0)),
            pl.BlockSpec((D, D), lambda i, *_: (0, 0)),
            pl.BlockSpec((D, D), lambda i, *_: (0, 0)),
        ],
        out_specs=[
            pl.BlockSpec((2, BS), lambda i, *_: (0, i)),
            pl.BlockSpec((1, BS), lambda i, *_: (0, i)),
            pl.BlockSpec((1, BS), lambda i, *_: (0, i)),
        ],
        scratch_shapes=[pltpu.VMEM((1, D), jnp.bfloat16)],
    )
    prob_t, mask_t, sel_t = pl.pallas_call(
        functools.partial(_routing_body, block_rows=BS),
        grid_spec=grid_spec,
        out_shape=[
            jax.ShapeDtypeStruct((2, T), jnp.float32),
            jax.ShapeDtypeStruct((1, T), jnp.float32),
            jax.ShapeDtypeStruct((1, T), jnp.float32),
        ],
        compiler_params=pltpu.CompilerParams(
            dimension_semantics=("arbitrary",)),
    )(cu_seqlens, tb, hidden_states, Wq, Wk)
    return (prob_t.T, mask_t.reshape(T).astype(bool), sel_t.reshape(T, 1))
